# baseline (speedup 1.0000x reference)
import functools

import jax
import jax.numpy as jnp
from jax.experimental import pallas as pl
from jax.experimental.pallas import tpu as pltpu


def _pre_body(w1_ref, w2_ref, b1_ref, b2_ref, v_ref, s_ref):
    h = pl.program_id(1)
    w2b = w2_ref[0]
    w2s = jnp.sum(w2b, axis=1, keepdims=True)
    pv = jax.lax.dot_general(w1_ref[0], w2s, (((1,), (0,)), ((), ())),
                             preferred_element_type=jnp.float32)
    ps = jax.lax.dot_general(b1_ref[0], w2s, (((1,), (0,)), ((), ())),
                             preferred_element_type=jnp.float32)

    @pl.when(h == 0)
    def _():
        v_ref[0] = pv
        s_ref[0] = ps + jnp.sum(b2_ref[0], axis=1, keepdims=True)

    @pl.when(h != 0)
    def _():
        v_ref[0] += pv
        s_ref[0] += ps


def _moe_body(tb, cap, x_ref, w_ref, p_ref, o_ref, tri_ref, carry_ref):
    i = pl.program_id(0)

    @pl.when(i == 0)
    def _():
        carry_ref[0] = 0
        rows = jax.lax.broadcasted_iota(jnp.int32, (tb, tb), 0)
        cols = jax.lax.broadcasted_iota(jnp.int32, (tb, tb), 1)
        tri_ref[...] = (cols <= rows).astype(jnp.bfloat16)

    proj = jax.lax.dot_general(x_ref[...], w_ref[...], (((1,), (0,)), ((), ())),
                               preferred_element_type=jnp.float32)
    l0 = proj[:, 0:1]
    l1 = proj[:, 1:2]
    is1 = l1 > l0
    gate = jax.nn.sigmoid(jnp.abs(l1 - l0))
    ind1 = is1.astype(jnp.float32)

    c1 = jax.lax.dot_general(tri_ref[...], ind1.astype(jnp.bfloat16),
                             (((1,), (0,)), ((), ())),
                             preferred_element_type=jnp.float32)
    cnt1 = c1 + carry_ref[0].astype(jnp.float32)
    gcnt = (jax.lax.broadcasted_iota(jnp.int32, (tb, 1), 0).astype(jnp.float32)
            + jnp.float32(1.0) + (i * tb).astype(jnp.float32))
    pos = jnp.where(is1, cnt1 - 1.0, gcnt - cnt1 - 1.0)
    keep = (pos < jnp.float32(cap)).astype(jnp.float32)

    dsel = (jnp.where(is1, proj[:, 3:4], proj[:, 2:3])
            + jnp.where(is1, p_ref[0:1, 1:2], p_ref[0:1, 0:1]))
    o_ref[...] = gate * keep * dsel
    carry_ref[0] += jnp.sum(ind1).astype(jnp.int32)


def _lsm_body(z_ref, o_ref):
    z = z_ref[...]
    m = jnp.max(z, axis=1, keepdims=True)
    lse = m + jnp.log(jnp.sum(jnp.exp(z - m), axis=1, keepdims=True))
    o_ref[...] = z - lse


def kernel(input, wg, w1, b1, w2, b2):
    B, S, D = input.shape
    E = wg.shape[1]
    H = w1.shape[2]
    T = B * S
    cap = (T + E - 1) // E
    f32 = jnp.float32

    HB = 512
    v, s = pl.pallas_call(
        _pre_body,
        grid=(E, H // HB),
        in_specs=[
            pl.BlockSpec((1, D, HB), lambda e, h: (e, 0, h)),
            pl.BlockSpec((1, HB, D), lambda e, h: (e, h, 0)),
            pl.BlockSpec((1, 1, HB), lambda e, h: (e, 0, h)),
            pl.BlockSpec((1, 1, D), lambda e, h: (e, 0, 0)),
        ],
        out_specs=[
            pl.BlockSpec((1, D, 1), lambda e, h: (e, 0, 0)),
            pl.BlockSpec((1, 1, 1), lambda e, h: (e, 0, 0)),
        ],
        out_shape=[
            jax.ShapeDtypeStruct((E, D, 1), f32),
            jax.ShapeDtypeStruct((E, 1, 1), f32),
        ],
    )(w1, w2, b1.reshape(E, 1, H), b2.reshape(E, 1, D))

    wcat = jnp.concatenate([wg, v[:, :, 0].T], axis=1)
    pvec = jnp.zeros((8, 128), f32).at[0, :E].set(s[:, 0, 0])
    xf = input.reshape(T, D)

    TB = 1024
    z = pl.pallas_call(
        functools.partial(_moe_body, TB, cap),
        grid=(T // TB,),
        in_specs=[
            pl.BlockSpec((TB, D), lambda i: (i, 0)),
            pl.BlockSpec((D, 2 * E), lambda i: (0, 0)),
            pl.BlockSpec((8, 128), lambda i: (0, 0)),
        ],
        out_specs=pl.BlockSpec((TB, 1), lambda i: (i, 0)),
        out_shape=jax.ShapeDtypeStruct((T, 1), f32),
        scratch_shapes=[
            pltpu.VMEM((TB, TB), jnp.bfloat16),
            pltpu.SMEM((1,), jnp.int32),
        ],
    )(xf, wcat, pvec)

    z2 = z.reshape(B, S)
    out = pl.pallas_call(
        _lsm_body,
        in_specs=[pl.BlockSpec((B, S), lambda: (0, 0))],
        out_specs=pl.BlockSpec((B, S), lambda: (0, 0)),
        out_shape=jax.ShapeDtypeStruct((B, S), f32),
    )(z2)
    return out


# speedup vs baseline: 1.0887x; 1.0887x over previous
import functools

import jax
import jax.numpy as jnp
from jax.experimental import pallas as pl
from jax.experimental.pallas import tpu as pltpu


def _fused_body(tb, cap, nh, n_e, pre,
                x_ref, x2_ref, wg_ref, w1_ref, w2_ref, b1_ref, b2_ref,
                o_ref, w8_ref, sv_ref, carry_ref):
    i = pl.program_id(0)

    @pl.when(i == 0)
    def _():
        carry_ref[0] = 0

    @pl.when(i < pre)
    def _():
        w2b = w2_ref[0]
        w2s = jnp.sum(w2b, axis=1, keepdims=True)
        pv = jax.lax.dot_general(w1_ref[0], w2s, (((1,), (0,)), ((), ())),
                                 preferred_element_type=jnp.float32)
        ps = jax.lax.dot_general(b1_ref[0], w2s, (((1,), (0,)), ((), ())),
                                 preferred_element_type=jnp.float32)
        e_idx = i // nh
        h_idx = i - e_idx * nh
        for e in range(n_e):
            c = n_e + e

            @pl.when(e_idx == e)
            def _():
                @pl.when(h_idx == 0)
                def _():
                    w8_ref[:, c:c + 1] = pv
                    sv_ref[0:1, e:e + 1] = (
                        ps + jnp.sum(b2_ref[0], axis=1, keepdims=True))

                @pl.when(h_idx != 0)
                def _():
                    w8_ref[:, c:c + 1] += pv
                    sv_ref[0:1, e:e + 1] += ps

        @pl.when(i == 0)
        def _():
            w8_ref[:, 0:n_e] = wg_ref[...]

    @pl.when(i >= pre)
    def _():
        j = i - pre

        d2 = x_ref.shape[1]
        proj = (jax.lax.dot_general(x_ref[...], w8_ref[0:d2, :],
                                    (((1,), (0,)), ((), ())),
                                    preferred_element_type=jnp.float32)
                + jax.lax.dot_general(x2_ref[...], w8_ref[d2:2 * d2, :],
                                      (((1,), (0,)), ((), ())),
                                      preferred_element_type=jnp.float32))
        l0 = proj[:, 0:1]
        l1 = proj[:, 1:2]
        is1 = l1 > l0
        gate = jax.nn.sigmoid(jnp.abs(l1 - l0))
        ind1 = is1.astype(jnp.float32)

        rows = jax.lax.broadcasted_iota(jnp.int32, (tb, tb), 0)
        cols = jax.lax.broadcasted_iota(jnp.int32, (tb, tb), 1)
        tri = (cols <= rows).astype(jnp.float32)
        c1 = jax.lax.dot_general(tri, ind1, (((1,), (0,)), ((), ())),
                                 preferred_element_type=jnp.float32)
        cnt1 = c1 + carry_ref[0].astype(jnp.float32)
        gcnt = (jax.lax.broadcasted_iota(jnp.int32, (tb, 1), 0).astype(jnp.float32)
                + jnp.float32(1.0) + (j * tb).astype(jnp.float32))
        pos = jnp.where(is1, cnt1 - 1.0, gcnt - cnt1 - 1.0)
        keep = (pos < jnp.float32(cap)).astype(jnp.float32)

        dsel = (jnp.where(is1, proj[:, 3:4], proj[:, 2:3])
                + jnp.where(is1, sv_ref[0:1, 1:2], sv_ref[0:1, 0:1]))
        o_ref[...] = gate * keep * dsel
        carry_ref[0] += jnp.sum(ind1).astype(jnp.int32)


def _lsm_body(z_ref, o_ref):
    z = z_ref[...]
    m = jnp.max(z, axis=1, keepdims=True)
    lse = m + jnp.log(jnp.sum(jnp.exp(z - m), axis=1, keepdims=True))
    o_ref[...] = z - lse


def kernel(input, wg, w1, b1, w2, b2):
    B, S, D = input.shape
    E = wg.shape[1]
    H = w1.shape[2]
    T = B * S
    cap = (T + E - 1) // E
    f32 = jnp.float32

    HB = 512
    TB = 1024
    NH = H // HB
    PRE = E * NH
    NB = T // TB
    xf = input.reshape(T, D)

    z = pl.pallas_call(
        functools.partial(_fused_body, TB, cap, NH, E, PRE),
        grid=(PRE + NB,),
        in_specs=[
            pl.BlockSpec((TB, D // 2), lambda i: (jnp.maximum(i - PRE, 0), 0)),
            pl.BlockSpec((TB, D // 2), lambda i: (jnp.maximum(i - PRE, 0), 1)),
            pl.BlockSpec((D, E), lambda i: (0, 0)),
            pl.BlockSpec((1, D, HB),
                         lambda i: (jnp.where(i < PRE, i // NH, E - 1), 0,
                                    jnp.where(i < PRE, i % NH, NH - 1))),
            pl.BlockSpec((1, HB, D),
                         lambda i: (jnp.where(i < PRE, i // NH, E - 1),
                                    jnp.where(i < PRE, i % NH, NH - 1), 0)),
            pl.BlockSpec((1, 1, HB),
                         lambda i: (jnp.where(i < PRE, i // NH, E - 1), 0,
                                    jnp.where(i < PRE, i % NH, NH - 1))),
            pl.BlockSpec((1, 1, D),
                         lambda i: (jnp.where(i < PRE, i // NH, E - 1), 0, 0)),
        ],
        out_specs=pl.BlockSpec((TB, 1), lambda i: (jnp.maximum(i - PRE, 0), 0)),
        out_shape=jax.ShapeDtypeStruct((T, 1), f32),
        scratch_shapes=[
            pltpu.VMEM((D, 2 * E), f32),
            pltpu.VMEM((8, 128), f32),
            pltpu.SMEM((1,), jnp.int32),
        ],
    )(xf, xf, wg, w1, w2, b1.reshape(E, 1, H), b2.reshape(E, 1, D))

    z2 = z.reshape(B, S)
    out = pl.pallas_call(
        _lsm_body,
        in_specs=[pl.BlockSpec((B, S), lambda: (0, 0))],
        out_specs=pl.BlockSpec((B, S), lambda: (0, 0)),
        out_shape=jax.ShapeDtypeStruct((B, S), f32),
    )(z2)
    return out
